# baseline (speedup 1.0000x reference)
import functools

import jax
import jax.numpy as jnp
from jax import lax
from jax.experimental import pallas as pl
from jax.experimental.pallas import tpu as pltpu
from jax.experimental.pallas import tpu_sc as plsc

BATCH = 16384
EMBED_DIM = 128
NC = 2
NS = 16
NW = NC * NS
TOK_PER_W = BATCH // NW
CHUNKS = (64, 64, 128, 128, 128)
OFFS = (0, 64, 128, 256, 384)
NCHUNK = len(CHUNKS)
KV = EMBED_DIM // 16
NBUF = 3
BUF_CAP = 128


def _dot_chunk(u_rows, i_rows, out_v, off, n, lanes):
    def group_body(g, _):
        t0 = g * 16

        def tok_body(tt, vec):
            t = t0 + tt
            prods = [u_rows[t, pl.ds(k * 16, 16)] * i_rows[t, pl.ds(k * 16, 16)]
                     for k in range(KV)]
            while len(prods) > 1:
                prods = [prods[p] + prods[p + 1]
                         for p in range(0, len(prods) - 1, 2)] + (
                            [prods[-1]] if len(prods) % 2 else [])
            s = jnp.sum(prods[0])
            return jnp.where(lanes == tt, s, vec)

        vec = lax.fori_loop(0, 16, tok_body, jnp.zeros((16,), jnp.float32),
                            unroll=2)
        out_v[pl.ds(off + t0, 16)] = vec
        return 0

    lax.fori_loop(0, n // 16, group_body, 0)


@functools.partial(
    pl.kernel,
    mesh=plsc.VectorSubcoreMesh(core_axis_name="c", subcore_axis_name="s"),
    out_type=jax.ShapeDtypeStruct((BATCH,), jnp.float32),
    compiler_params=pltpu.CompilerParams(needs_layout_passes=False),
    scratch_types=[
        pltpu.VMEM((TOK_PER_W,), jnp.int32),
        pltpu.VMEM((TOK_PER_W,), jnp.int32),
        pltpu.VMEM((NBUF, BUF_CAP, EMBED_DIM), jnp.float32),
        pltpu.VMEM((NBUF, BUF_CAP, EMBED_DIM), jnp.float32),
        pltpu.VMEM((TOK_PER_W,), jnp.float32),
        pltpu.SemaphoreType.DMA,
        pltpu.SemaphoreType.DMA,
        pltpu.SemaphoreType.DMA,
    ],
)
def _sc_dot(u_idx_hbm, i_idx_hbm, u_tab, i_tab, out_hbm,
            u_idx_v, i_idx_v, u_rows3, i_rows3, out_v, sem0, sem1, sem2):
    c = lax.axis_index("c")
    s = lax.axis_index("s")
    wid = s * NC + c
    sems = (sem0, sem1, sem2)

    cu_idx = pltpu.async_copy(
        u_idx_hbm.at[pl.ds(wid * TOK_PER_W, TOK_PER_W)], u_idx_v, sem0)
    ci_idx = pltpu.async_copy(
        i_idx_hbm.at[pl.ds(wid * TOK_PER_W, TOK_PER_W)], i_idx_v, sem1)
    cu_idx.wait()
    ci_idx.wait()

    def start(j):
        b = j % NBUF
        off, n = OFFS[j], CHUNKS[j]
        cu = pltpu.async_copy(
            u_tab.at[u_idx_v.at[pl.ds(off, n)]],
            u_rows3.at[b].at[pl.ds(0, n)], sems[b])
        ci = pltpu.async_copy(
            i_tab.at[i_idx_v.at[pl.ds(off, n)]],
            i_rows3.at[b].at[pl.ds(0, n)], sems[b])
        return cu, ci

    lanes = lax.iota(jnp.int32, 16)
    pend = {0: start(0), 1: start(1)}
    for j in range(NCHUNK):
        if j + 2 < NCHUNK:
            pend[j + 2] = start(j + 2)
        cu, ci = pend.pop(j)
        cu.wait()
        ci.wait()
        b = j % NBUF
        _dot_chunk(u_rows3.at[b], i_rows3.at[b], out_v, OFFS[j], CHUNKS[j],
                   lanes)

    pltpu.sync_copy(out_v, out_hbm.at[pl.ds(wid * TOK_PER_W, TOK_PER_W)])


def kernel(users, items, users_embedding, items_embedding):
    return _sc_dot(users, items, users_embedding, items_embedding)


# speedup vs baseline: 1.0310x; 1.0310x over previous
import functools

import jax
import jax.numpy as jnp
from jax import lax
from jax.experimental import pallas as pl
from jax.experimental.pallas import tpu as pltpu
from jax.experimental.pallas import tpu_sc as plsc

BATCH = 16384
EMBED_DIM = 128
NC = 2
NS = 16
NW = NC * NS
TOK_PER_W = BATCH // NW
CHUNK = 64
NCHUNK = TOK_PER_W // CHUNK
GROUPS = CHUNK // 16
KV = EMBED_DIM // 16
NBUF = 3


def _dot_chunk(u_rows, i_rows, out_v, j, lanes):
    def group_body(g, _):
        t0 = g * 16

        def tok_body(tt, vec):
            t = t0 + tt
            prods = [u_rows[t, pl.ds(k * 16, 16)] * i_rows[t, pl.ds(k * 16, 16)]
                     for k in range(KV)]
            while len(prods) > 1:
                prods = [prods[p] + prods[p + 1]
                         for p in range(0, len(prods) - 1, 2)] + (
                            [prods[-1]] if len(prods) % 2 else [])
            s = jnp.sum(prods[0])
            return jnp.where(lanes == tt, s, vec)

        vec = lax.fori_loop(0, 16, tok_body, jnp.zeros((16,), jnp.float32),
                            unroll=2)
        out_v[pl.ds(j * CHUNK + t0, 16)] = vec
        return 0

    lax.fori_loop(0, GROUPS, group_body, 0)


@functools.partial(
    pl.kernel,
    mesh=plsc.VectorSubcoreMesh(core_axis_name="c", subcore_axis_name="s"),
    out_type=jax.ShapeDtypeStruct((BATCH,), jnp.float32),
    compiler_params=pltpu.CompilerParams(needs_layout_passes=False),
    scratch_types=[
        pltpu.VMEM((TOK_PER_W,), jnp.int32),
        pltpu.VMEM((TOK_PER_W,), jnp.int32),
        pltpu.VMEM((NBUF, CHUNK, EMBED_DIM), jnp.float32),
        pltpu.VMEM((NBUF, CHUNK, EMBED_DIM), jnp.float32),
        pltpu.VMEM((TOK_PER_W,), jnp.float32),
        pltpu.SemaphoreType.DMA,
        pltpu.SemaphoreType.DMA,
        pltpu.SemaphoreType.DMA,
    ],
)
def _sc_dot(u_idx_hbm, i_idx_hbm, u_tab, i_tab, out_hbm,
            u_idx_v, i_idx_v, u_rows3, i_rows3, out_v, sem0, sem1, sem2):
    c = lax.axis_index("c")
    s = lax.axis_index("s")
    wid = s * NC + c
    sems = (sem0, sem1, sem2)

    cu_idx = pltpu.async_copy(
        u_idx_hbm.at[pl.ds(wid * TOK_PER_W, TOK_PER_W)], u_idx_v, sem0)
    ci_idx = pltpu.async_copy(
        i_idx_hbm.at[pl.ds(wid * TOK_PER_W, TOK_PER_W)], i_idx_v, sem1)
    cu_idx.wait()
    ci_idx.wait()

    def start(j):
        b = j % NBUF
        cu = pltpu.async_copy(
            u_tab.at[u_idx_v.at[pl.ds(j * CHUNK, CHUNK)]],
            u_rows3.at[b], sems[b])
        ci = pltpu.async_copy(
            i_tab.at[i_idx_v.at[pl.ds(j * CHUNK, CHUNK)]],
            i_rows3.at[b], sems[b])
        return cu, ci

    lanes = lax.iota(jnp.int32, 16)
    pend = {0: start(0), 1: start(1)}
    for j in range(NCHUNK):
        if j + 2 < NCHUNK:
            pend[j + 2] = start(j + 2)
        cu, ci = pend.pop(j)
        cu.wait()
        ci.wait()
        b = j % NBUF
        _dot_chunk(u_rows3.at[b], i_rows3.at[b], out_v, j, lanes)

    pltpu.sync_copy(out_v, out_hbm.at[pl.ds(wid * TOK_PER_W, TOK_PER_W)])


def kernel(users, items, users_embedding, items_embedding):
    return _sc_dot(users, items, users_embedding, items_embedding)
